# baseline (speedup 1.0000x reference)
import functools
import math

import jax
import jax.numpy as jnp
from jax import lax
from jax.experimental import pallas as pl
from jax.experimental.pallas import tpu as pltpu
from jax.experimental.pallas import tpu_sc as plsc

EMBED = 128
NREL = 16
NNODES = 10000
NEDGES = 320000
L = 16
BLK = 128
NBLK = NEDGES // BLK
NC, NS = 2, 16
NW = NC * NS
KMAX = (NBLK + NW - 1) // NW

_mesh = plsc.VectorSubcoreMesh(
    core_axis_name="c", subcore_axis_name="s", num_cores=NC, num_subcores=NS
)


@functools.partial(
    pl.kernel,
    out_type=jax.ShapeDtypeStruct((NEDGES,), jnp.float32),
    mesh=_mesh,
    scratch_types=[
        pltpu.VMEM((BLK,), jnp.int32),
        pltpu.VMEM((BLK,), jnp.int32),
        pltpu.VMEM((BLK,), jnp.int32),
        pltpu.VMEM((BLK, EMBED), jnp.float32),
        pltpu.VMEM((2 * L,), jnp.float32),
        pltpu.VMEM((BLK,), jnp.float32),
        pltpu.VMEM((BLK,), jnp.float32),
        pltpu.SemaphoreType.DMA,
    ],
    compiler_params=pltpu.CompilerParams(needs_layout_passes=False),
)
def _sc_scores(table, hidx, tidx, ridx, params, out,
               hidx_v, tidx_v, ridx_v, urows, par_v, ssq_v, out_v, sem):
    wid = lax.axis_index("s") * NC + lax.axis_index("c")

    pltpu.sync_copy(params, par_v)
    scale = par_v[pl.ds(0, L)]
    off = par_v[pl.ds(L, L)]
    lanes = lax.iota(jnp.int32, L)
    lane0 = lanes == 0

    def block(k, carry):
        j = wid + NW * k

        @pl.when(j < NBLK)
        def _():
            o = j * BLK
            pltpu.sync_copy(hidx.at[pl.ds(o, BLK)], hidx_v)
            pltpu.sync_copy(tidx.at[pl.ds(o, BLK)], tidx_v)
            pltpu.sync_copy(ridx.at[pl.ds(o, BLK)], ridx_v)
            pltpu.async_copy(table.at[hidx_v], urows, sem).wait()
            ct = pltpu.async_copy(table.at[tidx_v], urows, sem, add=True)
            cr = pltpu.async_copy(table.at[ridx_v], urows, sem, add=True)
            ct.wait()
            cr.wait()

            def edge(e, c2):
                acc = jnp.zeros((L,), jnp.float32)
                for c in range(EMBED // L):
                    u = urows[e, pl.ds(c * L, L)]
                    acc = acc + u * u
                s = jnp.sum(acc)
                plsc.store_scatter(
                    ssq_v, [jnp.full((L,), e, jnp.int32)],
                    jnp.broadcast_to(s, (L,)), mask=lane0)
                return c2

            lax.fori_loop(0, BLK, edge, 0)

            def grp(g, c2):
                x = ssq_v[pl.ds(g * L, L)]
                i = lax.bitcast_convert_type(x, jnp.int32)
                y = lax.bitcast_convert_type(
                    lax.shift_right_logical(i, 1) + jnp.int32(0x1FBD1DF5),
                    jnp.float32,
                )
                for _ in range(3):
                    y = 0.5 * (y + x / y)
                out_v[pl.ds(g * L, L)] = off - scale * y
                return c2

            lax.fori_loop(0, BLK // L, grp, 0)
            pltpu.sync_copy(out_v, out.at[pl.ds(o, BLK)])

        return carry

    lax.fori_loop(0, KMAX, block, 0)


def kernel(node_embeddings, edge_index, relation_type, rel_emb, temperature, bias):
    table = jnp.concatenate([node_embeddings, -node_embeddings, rel_emb], axis=0)
    hidx = edge_index[0].astype(jnp.int32)
    tidx = edge_index[1].astype(jnp.int32) + NNODES
    ridx = relation_type.astype(jnp.int32) + 2 * NNODES
    scale = (1.0 / (temperature * math.sqrt(EMBED))).astype(jnp.float32)
    off = (bias / temperature).astype(jnp.float32)
    params = jnp.concatenate(
        [jnp.broadcast_to(scale, (L,)), jnp.broadcast_to(off, (L,))]
    )
    return _sc_scores(table, hidx, tidx, ridx, params)


# speedup vs baseline: 1.0469x; 1.0469x over previous
import functools
import math

import jax
import jax.numpy as jnp
from jax import lax
from jax.experimental import pallas as pl
from jax.experimental.pallas import tpu as pltpu
from jax.experimental.pallas import tpu_sc as plsc

EMBED = 128
NREL = 16
NNODES = 10000
NEDGES = 320000
L = 16
BLK = 128
NBLK = NEDGES // BLK
NC, NS = 2, 16
NW = NC * NS
KMAX = (NBLK + NW - 1) // NW

_mesh = plsc.VectorSubcoreMesh(
    core_axis_name="c", subcore_axis_name="s", num_cores=NC, num_subcores=NS
)

_ROWS = lambda: pltpu.VMEM((BLK, EMBED), jnp.float32)


@functools.partial(
    pl.kernel,
    out_type=jax.ShapeDtypeStruct((NEDGES,), jnp.float32),
    mesh=_mesh,
    scratch_types=[
        [pltpu.VMEM((3, BLK), jnp.int32) for _ in range(2)],
        [_ROWS() for _ in range(2)],
        [_ROWS() for _ in range(2)],
        [_ROWS() for _ in range(2)],
        [pltpu.VMEM((BLK,), jnp.float32) for _ in range(2)],
        pltpu.VMEM((2 * L,), jnp.float32),
        pltpu.VMEM((BLK,), jnp.float32),
        [pltpu.SemaphoreType.DMA for _ in range(2)],
        [pltpu.SemaphoreType.DMA for _ in range(2)],
        [pltpu.SemaphoreType.DMA for _ in range(2)],
    ],
    compiler_params=pltpu.CompilerParams(needs_layout_passes=False),
)
def _sc_scores(table, idx3, params, out,
               idx_v, hrows, trows, rrows, out_v, par_v, ssq_v,
               semi, semg, semo):
    wid = lax.axis_index("s") * NC + lax.axis_index("c")

    pltpu.sync_copy(params, par_v)
    scale = par_v[pl.ds(0, L)]
    off = par_v[pl.ds(L, L)]
    lanes = lax.iota(jnp.int32, L)
    lane0 = lanes == 0

    def jof(k):
        return wid + NW * k

    def issue_idx(k, b):
        @pl.when((k >= 0) & (jof(k) < NBLK))
        def _():
            pltpu.async_copy(idx3.at[jof(k)], idx_v[b], semi[b])

    def drain_idx(k, b):
        @pl.when((k >= 0) & (jof(k) < NBLK))
        def _():
            pltpu.make_async_copy(idx3.at[0], idx_v[b], semi[b]).wait()

    def issue_gathers(k, b):
        @pl.when((k >= 0) & (jof(k) < NBLK))
        def _():
            pltpu.async_copy(table.at[idx_v[b].at[0]], hrows[b], semg[b])
            pltpu.async_copy(table.at[idx_v[b].at[1]], trows[b], semg[b])
            pltpu.async_copy(table.at[idx_v[b].at[2]], rrows[b], semg[b])

    def drain_gathers(k, b):
        @pl.when((k >= 0) & (jof(k) < NBLK))
        def _():
            pltpu.make_async_copy(table.at[pl.ds(0, BLK)], hrows[b], semg[b]).wait()
            pltpu.make_async_copy(table.at[pl.ds(0, BLK)], trows[b], semg[b]).wait()
            pltpu.make_async_copy(table.at[pl.ds(0, BLK)], rrows[b], semg[b]).wait()

    def drain_out(k, b):
        @pl.when((k >= 0) & (jof(k) < NBLK))
        def _():
            pltpu.make_async_copy(
                out_v[b], out.at[pl.ds(jof(k) * BLK, BLK)], semo[b]).wait()

    def compute(k, b):
        @pl.when((k >= 0) & (jof(k) < NBLK))
        def _():
            def edge(e, c2):
                acc = jnp.zeros((L,), jnp.float32)
                for c in range(EMBED // L):
                    u = (hrows[b][e, pl.ds(c * L, L)]
                         + rrows[b][e, pl.ds(c * L, L)]
                         - trows[b][e, pl.ds(c * L, L)])
                    acc = acc + u * u
                s = jnp.sum(acc)
                plsc.store_scatter(
                    ssq_v, [jnp.full((L,), e, jnp.int32)],
                    jnp.broadcast_to(s, (L,)), mask=lane0)
                return c2

            lax.fori_loop(0, BLK, edge, 0)

            def grp(g, c2):
                x = ssq_v[pl.ds(g * L, L)]
                i = lax.bitcast_convert_type(x, jnp.int32)
                y = lax.bitcast_convert_type(
                    lax.shift_right_logical(i, 1) + jnp.int32(0x1FBD1DF5),
                    jnp.float32,
                )
                for _ in range(3):
                    y = 0.5 * (y + x / y)
                out_v[b][pl.ds(g * L, L)] = off - scale * y
                return c2

            lax.fori_loop(0, BLK // L, grp, 0)
            pltpu.async_copy(out_v[b], out.at[pl.ds(jof(k) * BLK, BLK)], semo[b])

    def step(k, b):
        drain_out(k - 2, b)
        drain_gathers(k, b)
        drain_idx(k + 1, 1 - b)
        issue_gathers(k + 1, 1 - b)
        issue_idx(k + 2, b)
        compute(k, b)

    issue_idx(0, 0)
    drain_idx(0, 0)
    issue_gathers(0, 0)
    issue_idx(1, 1)

    npairs = KMAX // 2

    def pair(k2, carry):
        k = 2 * k2
        step(k, 0)
        step(k + 1, 1)
        return carry

    lax.fori_loop(0, npairs, pair, 0)
    if KMAX % 2:
        step(KMAX - 1, 0)
    drain_out(KMAX - 2, (KMAX - 2) % 2)
    drain_out(KMAX - 1, (KMAX - 1) % 2)


def kernel(node_embeddings, edge_index, relation_type, rel_emb, temperature, bias):
    table = jnp.concatenate([node_embeddings, rel_emb], axis=0)
    hidx = edge_index[0].astype(jnp.int32)
    tidx = edge_index[1].astype(jnp.int32)
    ridx = relation_type.astype(jnp.int32) + NNODES
    idx3 = jnp.stack(
        [hidx.reshape(NBLK, BLK), tidx.reshape(NBLK, BLK),
         ridx.reshape(NBLK, BLK)], axis=1)
    scale = (1.0 / (temperature * math.sqrt(EMBED))).astype(jnp.float32)
    off = (bias / temperature).astype(jnp.float32)
    params = jnp.concatenate(
        [jnp.broadcast_to(scale, (L,)), jnp.broadcast_to(off, (L,))]
    )
    return _sc_scores(table, idx3, params)
